# baseline (speedup 1.0000x reference)
import functools

import jax
import jax.numpy as jnp
from jax import lax
from jax.experimental import pallas as pl
from jax.experimental.pallas import tpu as pltpu
from jax.experimental.pallas import tpu_sc as plsc

N = 10000
E = 320000
D = 128
L = 16
NC = 2
NS = 16
NW = NC * NS
NPAD = 10240
RPS = NPAD // NS
CPS = NPAD // NW
EROWS = 2560
RPW = EROWS // NW
RPS2 = EROWS // NS
EPAD = EROWS * 128 - E

_mesh = plsc.VectorSubcoreMesh(core_axis_name="c", subcore_axis_name="s")
_sc_params = pltpu.CompilerParams(use_tc_tiling_on_sc=False)


def _fill_rows(ref, nrows, ncols, value):
    vec = jnp.full((L,), value, jnp.float32)

    def body(i, carry):
        for cb in range(ncols // L):
            ref[i, pl.ds(cb * L, L)] = vec
        return carry

    lax.fori_loop(0, nrows, body, 0)


def _fill_flat(ref, n, value):
    vec = jnp.full((L,), value, jnp.float32)

    def body(i, carry):
        ref[pl.ds(i * L, L)] = vec
        return carry

    lax.fori_loop(0, n // L, body, 0)


def _sc_agg_pipeline(table_hbm, src_hbm, dst_hbm, src_v, dst_v, buf_v,
                     sem_i, sem_g, sem_s, r0, nblocks, nb, sblk, acc_sh,
                     extra=None):
    nstg = nblocks // sblk
    pltpu.sync_copy(src_hbm.at[pl.ds(r0, sblk)], src_v.at[0])
    pltpu.sync_copy(dst_hbm.at[pl.ds(r0, sblk)], dst_v.at[0])
    gather = [None] * nblocks
    scat = [None] * nblocks
    degs = [None] * nblocks
    stage_pending = {}

    def stage(chunk):
        stage_pending[chunk] = [
            pltpu.async_copy(src_hbm.at[pl.ds(r0 + chunk * sblk, sblk)],
                             src_v.at[chunk % 2], sem_i),
            pltpu.async_copy(dst_hbm.at[pl.ds(r0 + chunk * sblk, sblk)],
                             dst_v.at[chunk % 2], sem_i),
        ]

    def issue_gather(j):
        chunk = j // sblk
        if chunk in stage_pending:
            for d_ in stage_pending.pop(chunk):
                d_.wait()
        gather[j] = pltpu.async_copy(
            table_hbm.at[src_v.at[chunk % 2, j % sblk]],
            buf_v.at[j % nb], sem_g)

    if nstg > 1:
        stage(1)
    for j in range(min(nb, nblocks)):
        issue_gather(j)
    for j in range(nblocks):
        gather[j].wait()
        chunk = j // sblk
        b = j % sblk
        scat[j] = pltpu.async_copy(buf_v.at[j % nb],
                                   acc_sh.at[dst_v.at[chunk % 2, b]],
                                   sem_s, add=True)
        if extra is not None:
            ones_v, deg_sh, sem_d = extra
            degs[j] = pltpu.async_copy(ones_v,
                                       deg_sh.at[dst_v.at[chunk % 2, b]],
                                       sem_d, add=True)
        nj = j + nb
        if nj < nblocks:
            scat[j].wait()
            scat[j] = None
            if degs[j] is not None:
                degs[j].wait()
                degs[j] = None
            if (j + 1) % sblk == 0:
                nchunk = (j + 1) // sblk + 1
                if nchunk < nstg:
                    stage(nchunk)
            issue_gather(nj)
    for j in range(nblocks):
        if scat[j] is not None:
            scat[j].wait()
        if degs[j] is not None:
            degs[j].wait()


@functools.partial(
    pl.kernel,
    mesh=_mesh,
    out_type=(
        jax.ShapeDtypeStruct((NC, NPAD, D), jnp.float32),
        jax.ShapeDtypeStruct((NC, NPAD), jnp.float32),
    ),
    scratch_types=[
        pltpu.VMEM((2, RPW // 10, 128), jnp.int32),
        pltpu.VMEM((2, RPW // 10, 128), jnp.int32),
        pltpu.VMEM((2, 128, D), jnp.float32),
        pltpu.VMEM((128,), jnp.float32),
        pltpu.VMEM((RPS,), jnp.float32),
        pltpu.SemaphoreType.DMA,
        pltpu.SemaphoreType.DMA,
        pltpu.SemaphoreType.DMA,
        pltpu.SemaphoreType.DMA,
        pltpu.VMEM_SHARED((NPAD, D), jnp.float32),
        pltpu.VMEM_SHARED((NPAD,), jnp.float32),
    ],
    compiler_params=_sc_params,
)
def _sc_layer1_agg(x_hbm, src_hbm, dst_hbm, p_out, deg_out,
                   src_v, dst_v, rows_v, ones_v, zd_v,
                   sem_i, sem_g, sem_s, sem_d, acc_sh, deg_sh):
    c = lax.axis_index("c")
    s = lax.axis_index("s")
    wid = s * NC + c
    r0 = wid * RPW

    _fill_rows(rows_v.at[0], 128, D, 0.0)
    _fill_flat(zd_v, RPS, 0.0)
    _fill_flat(ones_v, 128, 1.0)
    for k in range(RPS // 128):
        pltpu.sync_copy(rows_v.at[0], acc_sh.at[pl.ds(s * RPS + k * 128, 128)])
    pltpu.sync_copy(zd_v, deg_sh.at[pl.ds(s * RPS, RPS)])
    plsc.subcore_barrier()

    _sc_agg_pipeline(x_hbm, src_hbm, dst_hbm, src_v, dst_v, rows_v,
                     sem_i, sem_g, sem_s, r0, RPW, 2, RPW // 10, acc_sh,
                     extra=(ones_v, deg_sh, sem_d))
    plsc.subcore_barrier()

    for k in range(RPS // 128):
        sl = pl.ds(s * RPS + k * 128, 128)
        pltpu.sync_copy(acc_sh.at[sl], rows_v.at[0])
        pltpu.sync_copy(rows_v.at[0], p_out.at[c, sl])
    pltpu.sync_copy(deg_sh.at[pl.ds(s * RPS, RPS)], zd_v)
    pltpu.sync_copy(zd_v, deg_out.at[c, pl.ds(s * RPS, RPS)])


_NB2 = 8
_SBLK2 = 16


@functools.partial(
    pl.kernel,
    mesh=_mesh,
    out_type=jax.ShapeDtypeStruct((NC, NPAD, L), jnp.float32),
    scratch_types=[
        pltpu.VMEM((2, _SBLK2, 128), jnp.int32),
        pltpu.VMEM((2, _SBLK2, 128), jnp.int32),
        pltpu.VMEM((_NB2, 128, L), jnp.float32),
        pltpu.VMEM((128, L), jnp.float32),
        pltpu.SemaphoreType.DMA,
        pltpu.SemaphoreType.DMA,
        pltpu.SemaphoreType.DMA,
        pltpu.VMEM_SHARED((NPAD, L), jnp.float32),
    ],
    compiler_params=_sc_params,
)
def _sc_layer2_agg(s16_hbm, src_hbm, dst_hbm, s_out,
                   src_v, dst_v, vals_v, zb_v, sem_i, sem_g, sem_s, acc_sh):
    c = lax.axis_index("c")
    s = lax.axis_index("s")
    wid = s * NC + c
    r0 = wid * RPW

    _fill_rows(zb_v, 128, L, 0.0)
    for k in range(RPS // 128):
        pltpu.sync_copy(zb_v, acc_sh.at[pl.ds(s * RPS + k * 128, 128)])
    plsc.subcore_barrier()

    _sc_agg_pipeline(s16_hbm, src_hbm, dst_hbm, src_v, dst_v, vals_v,
                     sem_i, sem_g, sem_s, r0, RPW, _NB2, _SBLK2, acc_sh)
    plsc.subcore_barrier()

    for k in range(RPS // 128):
        sl = pl.ds(s * RPS + k * 128, 128)
        pltpu.sync_copy(acc_sh.at[sl], zb_v)
        pltpu.sync_copy(zb_v, s_out.at[c, sl])


_RB = 1000


def _tc_layer_body(x_ref, p0_ref, p1_ref, dd_ref,
                   ws1_ref, wn1_ref, b1_ref, wn2_ref,
                   h_ref, s_ref):
    hp = jax.lax.Precision.HIGHEST
    agg = (p0_ref[0] + p1_ref[0]) / dd_ref[...]
    h = (jnp.dot(x_ref[...], ws1_ref[...], precision=hp)
         + jnp.dot(agg, wn1_ref[...], precision=hp) + b1_ref[...])
    h = jax.nn.sigmoid(h)
    h_ref[...] = h
    s_ref[...] = jnp.broadcast_to(
        jnp.dot(h, wn2_ref[...], precision=hp), (_RB, L))


def _tc_layer(x, p, dd, ws1, wn1, b1, wn2):
    grid = (N // _RB,)
    row = lambda i: (i, 0)
    row3a = lambda i: (0, i, 0)
    row3b = lambda i: (1, i, 0)
    full = lambda i: (0, 0)
    return pl.pallas_call(
        _tc_layer_body,
        grid=grid,
        in_specs=[
            pl.BlockSpec((_RB, D), row),
            pl.BlockSpec((1, _RB, D), row3a),
            pl.BlockSpec((1, _RB, D), row3b),
            pl.BlockSpec((_RB, 1), row),
            pl.BlockSpec((D, D), full),
            pl.BlockSpec((D, D), full),
            pl.BlockSpec((1, D), full),
            pl.BlockSpec((D, 1), full),
        ],
        out_specs=[
            pl.BlockSpec((_RB, D), row),
            pl.BlockSpec((_RB, L), row),
        ],
        out_shape=[
            jax.ShapeDtypeStruct((N, D), jnp.float32),
            jax.ShapeDtypeStruct((N, L), jnp.float32),
        ],
    )(x, p, p, dd, ws1, wn1, b1, wn2)


def _tc_final_body(h_ref, s0_ref, s1_ref, dd_ref, ws2_ref, b2_ref, o_ref):
    hp = jax.lax.Precision.HIGHEST
    agg2 = (s0_ref[0, :, 0:1] + s1_ref[0, :, 0:1]) / dd_ref[...]
    o_ref[...] = (jnp.dot(h_ref[...], ws2_ref[...], precision=hp)
                  + b2_ref[...] + agg2)


def _tc_final(h, s2, dd, ws2, b2):
    grid = (N // _RB,)
    row = lambda i: (i, 0)
    row3a = lambda i: (0, i, 0)
    row3b = lambda i: (1, i, 0)
    full = lambda i: (0, 0)
    return pl.pallas_call(
        _tc_final_body,
        grid=grid,
        in_specs=[
            pl.BlockSpec((_RB, D), row),
            pl.BlockSpec((1, _RB, L), row3a),
            pl.BlockSpec((1, _RB, L), row3b),
            pl.BlockSpec((_RB, 1), row),
            pl.BlockSpec((D, 1), full),
            pl.BlockSpec((1, 1), full),
        ],
        out_specs=pl.BlockSpec((_RB, 1), row),
        out_shape=jax.ShapeDtypeStruct((N, 1), jnp.float32),
    )(h, s2, s2, dd, ws2, b2)


def kernel(x, edge_index, W_self1, W_neigh1, b1, W_self2, W_neigh2, b2):
    pad_i = jnp.arange(EPAD, dtype=jnp.int32)
    src2d = jnp.concatenate([edge_index[0], pad_i % 128]).reshape(EROWS, 128)
    dst2d = jnp.concatenate([edge_index[1], N + pad_i % (NPAD - N)]).reshape(EROWS, 128)

    p, deg = _sc_layer1_agg(x, src2d, dst2d)
    dd = jnp.maximum(deg[0] + deg[1], 1.0)[:N].reshape(N, 1)
    h, s16 = _tc_layer(x, p, dd, W_self1, W_neigh1, b1.reshape(1, D),
                       W_neigh2)
    s2 = _sc_layer2_agg(s16, src2d, dst2d)
    return _tc_final(h, s2, dd, W_self2, b2.reshape(1, 1))


# speedup vs baseline: 1.1543x; 1.1543x over previous
import functools

import jax
import jax.numpy as jnp
from jax import lax
from jax.experimental import pallas as pl
from jax.experimental.pallas import tpu as pltpu
from jax.experimental.pallas import tpu_sc as plsc

N = 10000
E = 320000
D = 128
L = 16
NC = 2
NS = 16
NW = NC * NS
NPAD = 10240
RPS = NPAD // NS
CPS = NPAD // NW
EROWS = 2560
RPW = EROWS // NW
RPS2 = EROWS // NS
EPAD = EROWS * 128 - E

_mesh = plsc.VectorSubcoreMesh(core_axis_name="c", subcore_axis_name="s")
_sc_params = pltpu.CompilerParams(use_tc_tiling_on_sc=False)


def _fill_rows(ref, nrows, ncols, value):
    vec = jnp.full((L,), value, jnp.float32)

    def body(i, carry):
        for cb in range(ncols // L):
            ref[i, pl.ds(cb * L, L)] = vec
        return carry

    lax.fori_loop(0, nrows, body, 0)


def _fill_flat(ref, n, value):
    vec = jnp.full((L,), value, jnp.float32)

    def body(i, carry):
        ref[pl.ds(i * L, L)] = vec
        return carry

    lax.fori_loop(0, n // L, body, 0)


def _sc_agg_pipeline(table_hbm, src_hbm, dst_hbm, src_v, dst_v, buf_v,
                     sem_i, sem_g, sem_s, r0, nblocks, nb, sblk, acc_sh,
                     extra=None):
    nstg = nblocks // sblk
    pltpu.sync_copy(src_hbm.at[pl.ds(r0, sblk)], src_v.at[0])
    pltpu.sync_copy(dst_hbm.at[pl.ds(r0, sblk)], dst_v.at[0])
    gather = [None] * nblocks
    scat = [None] * nblocks
    degs = [None] * nblocks
    stage_pending = {}

    def stage(chunk):
        stage_pending[chunk] = [
            pltpu.async_copy(src_hbm.at[pl.ds(r0 + chunk * sblk, sblk)],
                             src_v.at[chunk % 2], sem_i),
            pltpu.async_copy(dst_hbm.at[pl.ds(r0 + chunk * sblk, sblk)],
                             dst_v.at[chunk % 2], sem_i),
        ]

    def issue_gather(j):
        chunk = j // sblk
        if chunk in stage_pending:
            for d_ in stage_pending.pop(chunk):
                d_.wait()
        gather[j] = pltpu.async_copy(
            table_hbm.at[src_v.at[chunk % 2, j % sblk]],
            buf_v.at[j % nb], sem_g)

    if nstg > 1:
        stage(1)
    for j in range(min(nb, nblocks)):
        issue_gather(j)
    for j in range(nblocks):
        gather[j].wait()
        chunk = j // sblk
        b = j % sblk
        scat[j] = pltpu.async_copy(buf_v.at[j % nb],
                                   acc_sh.at[dst_v.at[chunk % 2, b]],
                                   sem_s, add=True)
        if extra is not None:
            ones_v, deg_sh, sem_d = extra
            degs[j] = pltpu.async_copy(ones_v,
                                       deg_sh.at[dst_v.at[chunk % 2, b]],
                                       sem_d, add=True)
        nj = j + nb
        if nj < nblocks:
            scat[j].wait()
            scat[j] = None
            if degs[j] is not None:
                degs[j].wait()
                degs[j] = None
            if (j + 1) % sblk == 0:
                nchunk = (j + 1) // sblk + 1
                if nchunk < nstg:
                    stage(nchunk)
            issue_gather(nj)
    for j in range(nblocks):
        if scat[j] is not None:
            scat[j].wait()
        if degs[j] is not None:
            degs[j].wait()


@functools.partial(
    pl.kernel,
    mesh=_mesh,
    out_type=(
        jax.ShapeDtypeStruct((NC, NPAD, D), jnp.float32),
        jax.ShapeDtypeStruct((NC, NPAD), jnp.float32),
    ),
    scratch_types=[
        pltpu.VMEM((2, RPW // 10, 128), jnp.int32),
        pltpu.VMEM((2, RPW // 10, 128), jnp.int32),
        pltpu.VMEM((2, 128, D), jnp.float32),
        pltpu.VMEM((128,), jnp.float32),
        pltpu.VMEM((RPS,), jnp.float32),
        pltpu.SemaphoreType.DMA,
        pltpu.SemaphoreType.DMA,
        pltpu.SemaphoreType.DMA,
        pltpu.SemaphoreType.DMA,
        pltpu.VMEM_SHARED((NPAD, D), jnp.float32),
        pltpu.VMEM_SHARED((NPAD,), jnp.float32),
    ],
    compiler_params=_sc_params,
)
def _sc_layer1_agg(x_hbm, src_hbm, dst_hbm, p_out, deg_out,
                   src_v, dst_v, rows_v, ones_v, zd_v,
                   sem_i, sem_g, sem_s, sem_d, acc_sh, deg_sh):
    c = lax.axis_index("c")
    s = lax.axis_index("s")
    wid = s * NC + c
    r0 = wid * RPW

    _fill_rows(rows_v.at[0], 128, D, 0.0)
    _fill_flat(zd_v, RPS, 0.0)
    _fill_flat(ones_v, 128, 1.0)
    for k in range(RPS // 128):
        pltpu.sync_copy(rows_v.at[0], acc_sh.at[pl.ds(s * RPS + k * 128, 128)])
    pltpu.sync_copy(zd_v, deg_sh.at[pl.ds(s * RPS, RPS)])
    plsc.subcore_barrier()

    _sc_agg_pipeline(x_hbm, src_hbm, dst_hbm, src_v, dst_v, rows_v,
                     sem_i, sem_g, sem_s, r0, RPW, 2, RPW // 10, acc_sh,
                     extra=(ones_v, deg_sh, sem_d))
    plsc.subcore_barrier()

    for k in range(RPS // 128):
        sl = pl.ds(s * RPS + k * 128, 128)
        pltpu.sync_copy(acc_sh.at[sl], rows_v.at[0])
        pltpu.sync_copy(rows_v.at[0], p_out.at[c, sl])
    pltpu.sync_copy(deg_sh.at[pl.ds(s * RPS, RPS)], zd_v)
    pltpu.sync_copy(zd_v, deg_out.at[c, pl.ds(s * RPS, RPS)])


_NB2 = 8
_SBLK2 = 16


@functools.partial(
    pl.kernel,
    mesh=_mesh,
    out_type=jax.ShapeDtypeStruct((NC, NPAD, L), jnp.float32),
    scratch_types=[
        pltpu.VMEM((2, _SBLK2, 128), jnp.int32),
        pltpu.VMEM((2, _SBLK2, 128), jnp.int32),
        pltpu.VMEM((_NB2, 128, L), jnp.float32),
        pltpu.VMEM((128, L), jnp.float32),
        pltpu.SemaphoreType.DMA,
        pltpu.SemaphoreType.DMA,
        pltpu.SemaphoreType.DMA,
        pltpu.VMEM_SHARED((NPAD, L), jnp.float32),
    ],
    compiler_params=_sc_params,
)
def _sc_layer2_agg(s16_hbm, src_hbm, dst_hbm, s_out,
                   src_v, dst_v, vals_v, zb_v, sem_i, sem_g, sem_s, acc_sh):
    c = lax.axis_index("c")
    s = lax.axis_index("s")
    wid = s * NC + c
    r0 = wid * RPW

    _fill_rows(zb_v, 128, L, 0.0)
    for k in range(RPS // 128):
        pltpu.sync_copy(zb_v, acc_sh.at[pl.ds(s * RPS + k * 128, 128)])
    plsc.subcore_barrier()

    _sc_agg_pipeline(s16_hbm, src_hbm, dst_hbm, src_v, dst_v, vals_v,
                     sem_i, sem_g, sem_s, r0, RPW, _NB2, _SBLK2, acc_sh)
    plsc.subcore_barrier()

    for k in range(RPS // 128):
        sl = pl.ds(s * RPS + k * 128, 128)
        pltpu.sync_copy(acc_sh.at[sl], zb_v)
        pltpu.sync_copy(zb_v, s_out.at[c, sl])


_RB = 1000


def _tc_layer_body(x_ref, p0_ref, p1_ref, dd_ref,
                   ws1_ref, wn1_ref, b1_ref, ws2_ref, wn2_ref, b2_ref,
                   s_ref, t_ref):
    agg = (p0_ref[0] + p1_ref[0]) / dd_ref[...]
    h = x_ref[...] @ ws1_ref[...] + agg @ wn1_ref[...] + b1_ref[...]
    h = jax.nn.sigmoid(h)
    s_ref[...] = jnp.broadcast_to(h @ wn2_ref[...], (_RB, L))
    t_ref[...] = h @ ws2_ref[...] + b2_ref[...]


def _tc_layer(x, p, dd, ws1, wn1, b1, ws2, wn2, b2):
    grid = (N // _RB,)
    row = lambda i: (i, 0)
    row3a = lambda i: (0, i, 0)
    row3b = lambda i: (1, i, 0)
    full = lambda i: (0, 0)
    return pl.pallas_call(
        _tc_layer_body,
        grid=grid,
        in_specs=[
            pl.BlockSpec((_RB, D), row),
            pl.BlockSpec((1, _RB, D), row3a),
            pl.BlockSpec((1, _RB, D), row3b),
            pl.BlockSpec((_RB, 1), row),
            pl.BlockSpec((D, D), full),
            pl.BlockSpec((D, D), full),
            pl.BlockSpec((1, D), full),
            pl.BlockSpec((D, 1), full),
            pl.BlockSpec((D, 1), full),
            pl.BlockSpec((1, 1), full),
        ],
        out_specs=[
            pl.BlockSpec((_RB, L), row),
            pl.BlockSpec((_RB, 1), row),
        ],
        out_shape=[
            jax.ShapeDtypeStruct((N, L), jnp.float32),
            jax.ShapeDtypeStruct((N, 1), jnp.float32),
        ],
    )(x, p, p, dd, ws1, wn1, b1, ws2, wn2, b2)


def _tc_final_body(t_ref, s0_ref, s1_ref, dd_ref, o_ref):
    o_ref[...] = t_ref[...] + (s0_ref[0, :, 0:1] + s1_ref[0, :, 0:1]) / dd_ref[...]


def _tc_final(t, s2, dd):
    grid = (N // _RB,)
    row = lambda i: (i, 0)
    row3a = lambda i: (0, i, 0)
    row3b = lambda i: (1, i, 0)
    return pl.pallas_call(
        _tc_final_body,
        grid=grid,
        in_specs=[
            pl.BlockSpec((_RB, 1), row),
            pl.BlockSpec((1, _RB, L), row3a),
            pl.BlockSpec((1, _RB, L), row3b),
            pl.BlockSpec((_RB, 1), row),
        ],
        out_specs=pl.BlockSpec((_RB, 1), row),
        out_shape=jax.ShapeDtypeStruct((N, 1), jnp.float32),
    )(t, s2, s2, dd)


def kernel(x, edge_index, W_self1, W_neigh1, b1, W_self2, W_neigh2, b2):
    pad_i = jnp.arange(EPAD, dtype=jnp.int32)
    src2d = jnp.concatenate([edge_index[0], pad_i % 128]).reshape(EROWS, 128)
    dst2d = jnp.concatenate([edge_index[1], N + pad_i % (NPAD - N)]).reshape(EROWS, 128)

    p, deg = _sc_layer1_agg(x, src2d, dst2d)
    dd = jnp.maximum(deg[0] + deg[1], 1.0)[:N].reshape(N, 1)
    s16, t = _tc_layer(x, p, dd, W_self1, W_neigh1, b1.reshape(1, D),
                       W_self2, W_neigh2, b2.reshape(1, 1))
    s2 = _sc_layer2_agg(s16, src2d, dst2d)
    return _tc_final(t, s2, dd)
